# Initial kernel scaffold; baseline (speedup 1.0000x reference)
import jax
import jax.numpy as jnp
from jax.experimental import pallas as pl


def kernel(h_src, h_dst, edge_attr_static, x_static_src, x_static_dst, edge_index, es_W1, es_b1, es_W2, es_b2, bw_W, bw_b, dg_W1, dg_b1, dg_W2, dg_b2, pl_W1, pl_b1, pl_W2, pl_b2):
    raise NotImplementedError("write your pallas kernel here")


# speedup vs baseline: 1.2039x; 1.2039x over previous
import functools

import jax
import jax.numpy as jnp
from jax import lax
from jax.experimental import pallas as pl
from jax.experimental.pallas import tpu as pltpu
from jax.experimental.pallas import tpu_sc as plsc

N = 10000
E = 320000
H = 128
DS = 16
DE = 16
HID = 128
MSG = 64

NC = 2
NS = 16
NW = NC * NS
EPW = E // NW
CH = 80
NCH = EPW // CH
SRCW = 384
DSTW = 256
NPAD = 10240
ROWS_PER_TILE = NPAD // NS

_mesh = plsc.VectorSubcoreMesh(core_axis_name="c", subcore_axis_name="s")


def _node_tables_body(hs, hd, xs, xd, dgW1, dgb1, esW1,
                      plW1, plb1, plW2, plb2, src_out, dst_out):
    f32 = jnp.float32
    a = jnp.dot(hs[...], dgW1[:H, :], preferred_element_type=f32) + dgb1[...]
    c = jnp.dot(xs[...], esW1[DE:DE + DS, :], preferred_element_type=f32)
    vh = jnp.maximum(
        jnp.dot(hs[...], plW1[...], preferred_element_type=f32) + plb1[...], 0.0)
    v = jnp.dot(vh, plW2[...], preferred_element_type=f32) + plb2[...]
    src_out[...] = jnp.concatenate(
        [a, c, v, jnp.zeros_like(v)], axis=1)
    b = jnp.dot(hd[...], dgW1[H:, :], preferred_element_type=f32)
    dd = jnp.dot(xd[...], esW1[DE + DS:, :], preferred_element_type=f32)
    dst_out[...] = jnp.concatenate([b, dd], axis=1)


def _node_tables(h_src, h_dst, xs, xd, dgW1, dgb1, esW1, plW1, plb1, plW2, plb2):
    nb = 2000
    grid = (N // nb,)
    row_spec = lambda w: pl.BlockSpec((nb, w), lambda i: (i, 0))
    full = lambda arr: pl.BlockSpec(arr.shape, lambda i: (0,) * arr.ndim)
    return pl.pallas_call(
        _node_tables_body,
        grid=grid,
        in_specs=[row_spec(H), row_spec(H), row_spec(DS), row_spec(DS),
                  full(dgW1), full(dgb1), full(esW1),
                  full(plW1), full(plb1), full(plW2), full(plb2)],
        out_specs=[row_spec(SRCW), row_spec(DSTW)],
        out_shape=[jax.ShapeDtypeStruct((N, SRCW), jnp.float32),
                   jax.ShapeDtypeStruct((N, DSTW), jnp.float32)],
    )(h_src, h_dst, xs, xd, dgW1, dgb1, esW1, plW1, plb1, plW2, plb2)


@functools.partial(
    pl.kernel,
    out_type=[jax.ShapeDtypeStruct((E, SRCW), jnp.float32),
              jax.ShapeDtypeStruct((E, DSTW), jnp.float32)],
    mesh=_mesh,
    scratch_types=[pltpu.VMEM((CH,), jnp.int32),
                   pltpu.VMEM((CH,), jnp.int32),
                   pltpu.VMEM((CH, SRCW), jnp.float32),
                   pltpu.VMEM((CH, DSTW), jnp.float32),
                   pltpu.SemaphoreType.DMA,
                   pltpu.SemaphoreType.DMA],
)
def _gather_kernel(srctab, dsttab, src_idx, dst_idx, sg_out, dg_out,
                   si_v, di_v, sr_v, dr_v, sem_s, sem_d):
    wid = lax.axis_index("s") * NC + lax.axis_index("c")
    base0 = wid * EPW

    def body(c, carry):
        base = base0 + c * CH
        pltpu.sync_copy(src_idx.at[pl.ds(base, CH)], si_v)
        pltpu.sync_copy(dst_idx.at[pl.ds(base, CH)], di_v)
        cp_s = pltpu.async_copy(srctab.at[si_v], sr_v, sem_s)
        cp_d = pltpu.async_copy(dsttab.at[di_v], dr_v, sem_d)
        cp_s.wait()
        cp_d.wait()
        pltpu.sync_copy(sr_v, sg_out.at[pl.ds(base, CH)])
        pltpu.sync_copy(dr_v, dg_out.at[pl.ds(base, CH)])
        return carry

    lax.fori_loop(0, NCH, body, 0)


def _edge_math_body(sg, dg, ea, dstev, dstod, esW1, esb1, esW2, esb2,
                    bwW, bwb, dgW2, dgb2, tgt_out, cm_out, *outs):
    f32 = jnp.float32
    eap = jnp.dot(ea[...], esW1[:DE, :], preferred_element_type=f32) + esb1[...]
    h1 = jnp.maximum(eap + sg[:, H:2 * H] + dg[:, H:2 * H], 0.0)
    wc = jnp.dot(esW2[...], bwW[...], preferred_element_type=f32)
    cb = jnp.dot(esb2[...], bwW[...], preferred_element_type=f32) + bwb[...]
    z = jnp.dot(h1, wc, preferred_element_type=f32) + cb
    be = jax.nn.softplus(z)
    gh = jnp.maximum(sg[:, :H] + dg[:, :H], 0.0)
    y = jnp.dot(gh, dgW2[...], preferred_element_type=f32) + dgb2[...]
    ge = jax.nn.sigmoid(y)
    msg = (be * ge) * sg[:, 2 * H:2 * H + MSG]
    for q in range(NPASS):
        outs[q][...] = msg[:, 8 * q:8 * q + 8]
    npair = dstev.shape[0]
    de = dstev[...]
    do = dstod[...]
    hi = lax.broadcasted_iota(jnp.int32, (npair, 16), 1) >= 8
    col = lax.broadcasted_iota(jnp.int32, (npair, 16), 1) - 8 * hi.astype(
        jnp.int32)
    tgt_out[...] = jnp.where(hi, do * 8, de * 8) + col
    cm_out[...] = (hi & (de == do)).astype(jnp.int32)


def _edge_math(sg, dg, ea, dstev, dstod,
               esW1, esb1, esW2, esb2, bwW, bwb, dgW2, dgb2):
    be_blk = 800
    npair = be_blk // 2
    grid = (E // be_blk,)
    row_spec = lambda w: pl.BlockSpec((be_blk, w), lambda i: (i, 0))
    full = lambda arr: pl.BlockSpec(arr.shape, lambda i: (0,) * arr.ndim)
    pair_spec = pl.BlockSpec((npair, 1), lambda i: (i, 0))
    pout_spec = pl.BlockSpec((npair, 16), lambda i: (i, 0))
    return pl.pallas_call(
        _edge_math_body,
        grid=grid,
        in_specs=[row_spec(SRCW), row_spec(DSTW), row_spec(DE),
                  pair_spec, pair_spec,
                  full(esW1), full(esb1), full(esW2), full(esb2),
                  full(bwW), full(bwb), full(dgW2), full(dgb2)],
        out_specs=[pout_spec, pout_spec] + [row_spec(8)] * NPASS,
        out_shape=[jax.ShapeDtypeStruct((E // 2, 16), jnp.int32),
                   jax.ShapeDtypeStruct((E // 2, 16), jnp.int32)]
        + [jax.ShapeDtypeStruct((E, 8), jnp.float32)] * NPASS,
        compiler_params=pltpu.CompilerParams(
            dimension_semantics=("arbitrary",)),
    )(sg, dg, ea, dstev, dstod,
      esW1, esb1, esW2, esb2, bwW, bwb, dgW2, dgb2)


CH2 = 400
NCH2 = EPW // CH2
NPASS = MSG // 8


@functools.partial(
    pl.kernel,
    out_type=jax.ShapeDtypeStruct((NW, NPASS, NPAD * 8), jnp.float32),
    mesh=_mesh,
    scratch_types=[pltpu.VMEM((CH2 * 8,), jnp.int32),
                   pltpu.VMEM((CH2 * 8,), jnp.int32),
                   pltpu.VMEM((CH2 * 8,), jnp.float32),
                   pltpu.VMEM((NPAD * 8,), jnp.float32)],
    compiler_params=pltpu.CompilerParams(needs_layout_passes=False),
)
def _scatter_kernel(m0, m1, m2, m3, m4, m5, m6, m7, tgt_hbm, cm_hbm,
                    zeros_hbm, out_hbm, tgt_v, cm_v, msg_v, acc):
    cid = lax.axis_index("c")
    sid = lax.axis_index("s")
    wid = cid * NS + sid
    base0 = wid * EPW
    msgs = (m0, m1, m2, m3, m4, m5, m6, m7)
    npair = CH2 // 2

    for p in range(NPASS):
        pltpu.sync_copy(zeros_hbm, acc)

        def chunk_body(c, carry, _mp=msgs[p]):
            base = base0 + c * CH2
            pltpu.sync_copy(tgt_hbm.at[pl.ds(base * 8, CH2 * 8)], tgt_v)
            pltpu.sync_copy(cm_hbm.at[pl.ds(base * 8, CH2 * 8)], cm_v)
            pltpu.sync_copy(_mp.at[pl.ds(base * 8, CH2 * 8)], msg_v)

            def pair_body(k, carry2):
                sl = pl.ds(16 * k, 16)
                tgt = tgt_v[sl]
                mask2 = cm_v[sl] > 0
                vals = msg_v[sl]
                plsc.addupdate_scatter(acc, [tgt], vals,
                                       mask=jnp.logical_not(mask2))
                plsc.addupdate_scatter(acc, [tgt], vals, mask=mask2)
                return carry2

            lax.fori_loop(0, npair, pair_body, 0)
            return carry

        lax.fori_loop(0, NCH2, chunk_body, 0)
        pltpu.sync_copy(acc, out_hbm.at[wid, p])


def _sum_body(p, out):
    out[...] = jnp.sum(p[...], axis=0)


def _sum_partials(partials):
    rows = NPASS * NPAD * 8 // 128
    p3 = partials.reshape(NW, rows, 128)
    blk = 512
    grid = (rows // blk,)
    red = pl.pallas_call(
        _sum_body,
        grid=grid,
        in_specs=[pl.BlockSpec((NW, blk, 128), lambda i: (0, i, 0))],
        out_specs=pl.BlockSpec((blk, 128), lambda i: (i, 0)),
        out_shape=jax.ShapeDtypeStruct((rows, 128), jnp.float32),
        compiler_params=pltpu.CompilerParams(
            dimension_semantics=("arbitrary",)),
    )(p3)
    r = red.reshape(NPASS, NPAD, 8)
    return jnp.transpose(r, (1, 0, 2)).reshape(NPAD, MSG)[:N]


def kernel(h_src, h_dst, edge_attr_static, x_static_src, x_static_dst,
           edge_index,
           es_W1, es_b1, es_W2, es_b2,
           bw_W, bw_b,
           dg_W1, dg_b1, dg_W2, dg_b2,
           pl_W1, pl_b1, pl_W2, pl_b2):
    f32 = jnp.float32
    src = edge_index[0].astype(jnp.int32)
    dst = edge_index[1].astype(jnp.int32)
    dgb1 = dg_b1.reshape(1, HID).astype(f32)
    esb1 = es_b1.reshape(1, HID).astype(f32)
    esb2 = es_b2.reshape(1, HID).astype(f32)
    plb1 = pl_b1.reshape(1, HID).astype(f32)
    plb2 = pl_b2.reshape(1, MSG).astype(f32)
    bwb = bw_b.reshape(1, 1).astype(f32)
    dgb2 = dg_b2.reshape(1, 1).astype(f32)

    srctab, dsttab = _node_tables(h_src, h_dst, x_static_src, x_static_dst,
                                  dg_W1, dgb1, es_W1, pl_W1, plb1, pl_W2, plb2)
    sg, dg = _gather_kernel(srctab, dsttab, src, dst)
    dstev = dst[0::2].reshape(E // 2, 1)
    dstod = dst[1::2].reshape(E // 2, 1)
    tgt16, cm16, *msgs = _edge_math(sg, dg, edge_attr_static, dstev, dstod,
                                    es_W1, esb1, es_W2, esb2, bw_W, bwb,
                                    dg_W2, dgb2)
    zeros = jnp.zeros((NPAD * 8,), f32)
    partials = _scatter_kernel(*[m.reshape(E * 8) for m in msgs],
                               tgt16.reshape(E * 8), cm16.reshape(E * 8),
                               zeros)
    return _sum_partials(partials.reshape(NW, NPASS * NPAD * 8))


# speedup vs baseline: 1.2626x; 1.0488x over previous
import functools

import jax
import jax.numpy as jnp
from jax import lax
from jax.experimental import pallas as pl
from jax.experimental.pallas import tpu as pltpu
from jax.experimental.pallas import tpu_sc as plsc

N = 10000
E = 320000
H = 128
DS = 16
DE = 16
HID = 128
MSG = 64

NC = 2
NS = 16
NW = NC * NS
EPW = E // NW
CH = 80
NCH = EPW // CH
SRCW = 384
DSTW = 256
NPAD = 10240
ROWS_PER_TILE = NPAD // NS

_mesh = plsc.VectorSubcoreMesh(core_axis_name="c", subcore_axis_name="s")


def _node_tables_body(hs, hd, xs, xd, dgW1, dgb1, esW1,
                      plW1, plb1, plW2, plb2, src_out, dst_out):
    f32 = jnp.float32
    a = jnp.dot(hs[...], dgW1[:H, :], preferred_element_type=f32) + dgb1[...]
    c = jnp.dot(xs[...], esW1[DE:DE + DS, :], preferred_element_type=f32)
    vh = jnp.maximum(
        jnp.dot(hs[...], plW1[...], preferred_element_type=f32) + plb1[...], 0.0)
    v = jnp.dot(vh, plW2[...], preferred_element_type=f32) + plb2[...]
    src_out[...] = jnp.concatenate(
        [a, c, v, jnp.zeros_like(v)], axis=1)
    b = jnp.dot(hd[...], dgW1[H:, :], preferred_element_type=f32)
    dd = jnp.dot(xd[...], esW1[DE + DS:, :], preferred_element_type=f32)
    dst_out[...] = jnp.concatenate([b, dd], axis=1)


def _node_tables(h_src, h_dst, xs, xd, dgW1, dgb1, esW1, plW1, plb1, plW2, plb2):
    nb = 2000
    grid = (N // nb,)
    row_spec = lambda w: pl.BlockSpec((nb, w), lambda i: (i, 0))
    full = lambda arr: pl.BlockSpec(arr.shape, lambda i: (0,) * arr.ndim)
    return pl.pallas_call(
        _node_tables_body,
        grid=grid,
        in_specs=[row_spec(H), row_spec(H), row_spec(DS), row_spec(DS),
                  full(dgW1), full(dgb1), full(esW1),
                  full(plW1), full(plb1), full(plW2), full(plb2)],
        out_specs=[row_spec(SRCW), row_spec(DSTW)],
        out_shape=[jax.ShapeDtypeStruct((N, SRCW), jnp.float32),
                   jax.ShapeDtypeStruct((N, DSTW), jnp.float32)],
    )(h_src, h_dst, xs, xd, dgW1, dgb1, esW1, plW1, plb1, plW2, plb2)


@functools.partial(
    pl.kernel,
    out_type=[jax.ShapeDtypeStruct((E, SRCW), jnp.float32),
              jax.ShapeDtypeStruct((E, DSTW), jnp.float32)],
    mesh=_mesh,
    scratch_types=[pltpu.VMEM((CH,), jnp.int32),
                   pltpu.VMEM((CH,), jnp.int32),
                   pltpu.VMEM((CH, SRCW), jnp.float32),
                   pltpu.VMEM((CH, DSTW), jnp.float32),
                   pltpu.SemaphoreType.DMA,
                   pltpu.SemaphoreType.DMA],
)
def _gather_kernel(srctab, dsttab, src_idx, dst_idx, sg_out, dg_out,
                   si_v, di_v, sr_v, dr_v, sem_s, sem_d):
    wid = lax.axis_index("s") * NC + lax.axis_index("c")
    base0 = wid * EPW

    def body(c, carry):
        base = base0 + c * CH
        pltpu.sync_copy(src_idx.at[pl.ds(base, CH)], si_v)
        pltpu.sync_copy(dst_idx.at[pl.ds(base, CH)], di_v)
        cp_s = pltpu.async_copy(srctab.at[si_v], sr_v, sem_s)
        cp_d = pltpu.async_copy(dsttab.at[di_v], dr_v, sem_d)
        cp_s.wait()
        cp_d.wait()
        pltpu.sync_copy(sr_v, sg_out.at[pl.ds(base, CH)])
        pltpu.sync_copy(dr_v, dg_out.at[pl.ds(base, CH)])
        return carry

    lax.fori_loop(0, NCH, body, 0)


def _edge_math_body(sg, dg, ea, dstev, dstod, esW1, esb1, esW2, esb2,
                    bwW, bwb, dgW2, dgb2, tgt_out, cm_out, *outs):
    f32 = jnp.float32
    eap = jnp.dot(ea[...], esW1[:DE, :], preferred_element_type=f32) + esb1[...]
    h1 = jnp.maximum(eap + sg[:, H:2 * H] + dg[:, H:2 * H], 0.0)
    wc = jnp.dot(esW2[...], bwW[...], preferred_element_type=f32)
    cb = jnp.dot(esb2[...], bwW[...], preferred_element_type=f32) + bwb[...]
    z = jnp.dot(h1, wc, preferred_element_type=f32) + cb
    be = jax.nn.softplus(z)
    gh = jnp.maximum(sg[:, :H] + dg[:, :H], 0.0)
    y = jnp.dot(gh, dgW2[...], preferred_element_type=f32) + dgb2[...]
    ge = jax.nn.sigmoid(y)
    msg = (be * ge) * sg[:, 2 * H:2 * H + MSG]
    for q in range(NPASS):
        outs[q][...] = msg[:, 8 * q:8 * q + 8]
    npair = dstev.shape[0]
    de = dstev[...]
    do = dstod[...]
    hi = lax.broadcasted_iota(jnp.int32, (npair, 16), 1) >= 8
    col = lax.broadcasted_iota(jnp.int32, (npair, 16), 1) - 8 * hi.astype(
        jnp.int32)
    tgt_out[...] = jnp.where(hi, do * 8, de * 8) + col
    cm_out[...] = (hi & (de == do)).astype(jnp.int32)


def _edge_math(sg, dg, ea, dstev, dstod,
               esW1, esb1, esW2, esb2, bwW, bwb, dgW2, dgb2):
    be_blk = 800
    npair = be_blk // 2
    grid = (E // be_blk,)
    row_spec = lambda w: pl.BlockSpec((be_blk, w), lambda i: (i, 0))
    full = lambda arr: pl.BlockSpec(arr.shape, lambda i: (0,) * arr.ndim)
    pair_spec = pl.BlockSpec((npair, 1), lambda i: (i, 0))
    pout_spec = pl.BlockSpec((npair, 16), lambda i: (i, 0))
    return pl.pallas_call(
        _edge_math_body,
        grid=grid,
        in_specs=[row_spec(SRCW), row_spec(DSTW), row_spec(DE),
                  pair_spec, pair_spec,
                  full(esW1), full(esb1), full(esW2), full(esb2),
                  full(bwW), full(bwb), full(dgW2), full(dgb2)],
        out_specs=[pout_spec, pout_spec] + [row_spec(8)] * NPASS,
        out_shape=[jax.ShapeDtypeStruct((E // 2, 16), jnp.int32),
                   jax.ShapeDtypeStruct((E // 2, 16), jnp.int32)]
        + [jax.ShapeDtypeStruct((E, 8), jnp.float32)] * NPASS,
        compiler_params=pltpu.CompilerParams(
            dimension_semantics=("arbitrary",)),
    )(sg, dg, ea, dstev, dstod,
      esW1, esb1, esW2, esb2, bwW, bwb, dgW2, dgb2)


CH2 = 400
NCH2 = EPW // CH2
NPASS = MSG // 8


@functools.partial(
    pl.kernel,
    out_type=jax.ShapeDtypeStruct((NW, NPASS, NPAD * 8), jnp.float32),
    mesh=_mesh,
    scratch_types=[pltpu.VMEM((CH2 * 8,), jnp.int32),
                   pltpu.VMEM((CH2 * 8,), jnp.int32),
                   pltpu.VMEM((CH2 * 8,), jnp.float32),
                   pltpu.VMEM((NPAD * 8,), jnp.float32)],
    compiler_params=pltpu.CompilerParams(needs_layout_passes=False),
)
def _scatter_kernel(m0, m1, m2, m3, m4, m5, m6, m7, tgt_hbm, cm_hbm,
                    zeros_hbm, out_hbm, tgt_v, cm_v, msg_v, acc):
    cid = lax.axis_index("c")
    sid = lax.axis_index("s")
    wid = cid * NS + sid
    base0 = wid * EPW
    msgs = (m0, m1, m2, m3, m4, m5, m6, m7)
    npair = CH2 // 2

    for p in range(NPASS):
        pltpu.sync_copy(zeros_hbm, acc)

        def chunk_body(c, carry, _mp=msgs[p]):
            base = base0 + c * CH2
            pltpu.sync_copy(tgt_hbm.at[pl.ds(base * 8, CH2 * 8)], tgt_v)
            pltpu.sync_copy(cm_hbm.at[pl.ds(base * 8, CH2 * 8)], cm_v)
            pltpu.sync_copy(_mp.at[pl.ds(base * 8, CH2 * 8)], msg_v)

            @plsc.parallel_loop(0, npair, 1, unroll=8)
            def _pair_body(k):
                sl = pl.ds(16 * k, 16)
                tgt = tgt_v[sl]
                mask2 = cm_v[sl] > 0
                vals = msg_v[sl]
                plsc.addupdate_scatter(acc, [tgt], vals,
                                       mask=jnp.logical_not(mask2))
                plsc.addupdate_scatter(acc, [tgt], vals, mask=mask2)

            return carry

        lax.fori_loop(0, NCH2, chunk_body, 0)
        pltpu.sync_copy(acc, out_hbm.at[wid, p])


def _sum_body(p, out):
    out[...] = jnp.sum(p[...], axis=0)


def _sum_partials(partials):
    rows = NPASS * NPAD * 8 // 128
    p3 = partials.reshape(NW, rows, 128)
    blk = 512
    grid = (rows // blk,)
    red = pl.pallas_call(
        _sum_body,
        grid=grid,
        in_specs=[pl.BlockSpec((NW, blk, 128), lambda i: (0, i, 0))],
        out_specs=pl.BlockSpec((blk, 128), lambda i: (i, 0)),
        out_shape=jax.ShapeDtypeStruct((rows, 128), jnp.float32),
        compiler_params=pltpu.CompilerParams(
            dimension_semantics=("arbitrary",)),
    )(p3)
    r = red.reshape(NPASS, NPAD, 8)
    return jnp.transpose(r, (1, 0, 2)).reshape(NPAD, MSG)[:N]


def kernel(h_src, h_dst, edge_attr_static, x_static_src, x_static_dst,
           edge_index,
           es_W1, es_b1, es_W2, es_b2,
           bw_W, bw_b,
           dg_W1, dg_b1, dg_W2, dg_b2,
           pl_W1, pl_b1, pl_W2, pl_b2):
    f32 = jnp.float32
    src = edge_index[0].astype(jnp.int32)
    dst = edge_index[1].astype(jnp.int32)
    dgb1 = dg_b1.reshape(1, HID).astype(f32)
    esb1 = es_b1.reshape(1, HID).astype(f32)
    esb2 = es_b2.reshape(1, HID).astype(f32)
    plb1 = pl_b1.reshape(1, HID).astype(f32)
    plb2 = pl_b2.reshape(1, MSG).astype(f32)
    bwb = bw_b.reshape(1, 1).astype(f32)
    dgb2 = dg_b2.reshape(1, 1).astype(f32)

    srctab, dsttab = _node_tables(h_src, h_dst, x_static_src, x_static_dst,
                                  dg_W1, dgb1, es_W1, pl_W1, plb1, pl_W2, plb2)
    sg, dg = _gather_kernel(srctab, dsttab, src, dst)
    dstev = dst[0::2].reshape(E // 2, 1)
    dstod = dst[1::2].reshape(E // 2, 1)
    tgt16, cm16, *msgs = _edge_math(sg, dg, edge_attr_static, dstev, dstod,
                                    es_W1, esb1, es_W2, esb2, bw_W, bwb,
                                    dg_W2, dgb2)
    zeros = jnp.zeros((NPAD * 8,), f32)
    partials = _scatter_kernel(*[m.reshape(E * 8) for m in msgs],
                               tgt16.reshape(E * 8), cm16.reshape(E * 8),
                               zeros)
    return _sum_partials(partials.reshape(NW, NPASS * NPAD * 8))


# speedup vs baseline: 1.3814x; 1.0941x over previous
import functools

import jax
import jax.numpy as jnp
from jax import lax
from jax.experimental import pallas as pl
from jax.experimental.pallas import tpu as pltpu
from jax.experimental.pallas import tpu_sc as plsc

N = 10000
E = 320000
H = 128
DS = 16
DE = 16
HID = 128
MSG = 64

NC = 2
NS = 16
NW = NC * NS
EPW = E // NW
CH = 80
NCH = EPW // CH
SRCW = 384
DSTW = 256
NPAD = 10240
ROWS_PER_TILE = NPAD // NS

_mesh = plsc.VectorSubcoreMesh(core_axis_name="c", subcore_axis_name="s")


def _node_tables_body(hs, hd, xs, xd, dgW1, dgb1, esW1,
                      plW1, plb1, plW2, plb2, src_out, dst_out):
    f32 = jnp.float32
    a = jnp.dot(hs[...], dgW1[:H, :], preferred_element_type=f32) + dgb1[...]
    c = jnp.dot(xs[...], esW1[DE:DE + DS, :], preferred_element_type=f32)
    vh = jnp.maximum(
        jnp.dot(hs[...], plW1[...], preferred_element_type=f32) + plb1[...], 0.0)
    v = jnp.dot(vh, plW2[...], preferred_element_type=f32) + plb2[...]
    src_out[...] = jnp.concatenate(
        [a, c, v, jnp.zeros_like(v)], axis=1)
    b = jnp.dot(hd[...], dgW1[H:, :], preferred_element_type=f32)
    dd = jnp.dot(xd[...], esW1[DE + DS:, :], preferred_element_type=f32)
    dst_out[...] = jnp.concatenate([b, dd], axis=1)


def _node_tables(h_src, h_dst, xs, xd, dgW1, dgb1, esW1, plW1, plb1, plW2, plb2):
    nb = 2000
    grid = (N // nb,)
    row_spec = lambda w: pl.BlockSpec((nb, w), lambda i: (i, 0))
    full = lambda arr: pl.BlockSpec(arr.shape, lambda i: (0,) * arr.ndim)
    return pl.pallas_call(
        _node_tables_body,
        grid=grid,
        in_specs=[row_spec(H), row_spec(H), row_spec(DS), row_spec(DS),
                  full(dgW1), full(dgb1), full(esW1),
                  full(plW1), full(plb1), full(plW2), full(plb2)],
        out_specs=[row_spec(SRCW), row_spec(DSTW)],
        out_shape=[jax.ShapeDtypeStruct((N, SRCW), jnp.float32),
                   jax.ShapeDtypeStruct((N, DSTW), jnp.float32)],
    )(h_src, h_dst, xs, xd, dgW1, dgb1, esW1, plW1, plb1, plW2, plb2)


@functools.partial(
    pl.kernel,
    out_type=[jax.ShapeDtypeStruct((E, SRCW), jnp.float32),
              jax.ShapeDtypeStruct((E, DSTW), jnp.float32)],
    mesh=_mesh,
    scratch_types=[pltpu.VMEM((CH,), jnp.int32),
                   pltpu.VMEM((CH,), jnp.int32),
                   pltpu.VMEM((CH, SRCW), jnp.float32),
                   pltpu.VMEM((CH, DSTW), jnp.float32),
                   pltpu.SemaphoreType.DMA,
                   pltpu.SemaphoreType.DMA],
)
def _gather_kernel(srctab, dsttab, src_idx, dst_idx, sg_out, dg_out,
                   si_v, di_v, sr_v, dr_v, sem_s, sem_d):
    wid = lax.axis_index("s") * NC + lax.axis_index("c")
    base0 = wid * EPW

    def body(c, carry):
        base = base0 + c * CH
        pltpu.sync_copy(src_idx.at[pl.ds(base, CH)], si_v)
        pltpu.sync_copy(dst_idx.at[pl.ds(base, CH)], di_v)
        cp_s = pltpu.async_copy(srctab.at[si_v], sr_v, sem_s)
        cp_d = pltpu.async_copy(dsttab.at[di_v], dr_v, sem_d)
        cp_s.wait()
        cp_d.wait()
        pltpu.sync_copy(sr_v, sg_out.at[pl.ds(base, CH)])
        pltpu.sync_copy(dr_v, dg_out.at[pl.ds(base, CH)])
        return carry

    lax.fori_loop(0, NCH, body, 0)


def _edge_math_body(sg, dg, ea, dstev, dstod, esW1, esb1, esW2, esb2,
                    bwW, bwb, dgW2, dgb2, tgt_out, cm_out, *outs):
    f32 = jnp.float32
    eap = jnp.dot(ea[...], esW1[:DE, :], preferred_element_type=f32) + esb1[...]
    h1 = jnp.maximum(eap + sg[:, H:2 * H] + dg[:, H:2 * H], 0.0)
    wc = jnp.dot(esW2[...], bwW[...], preferred_element_type=f32)
    cb = jnp.dot(esb2[...], bwW[...], preferred_element_type=f32) + bwb[...]
    z = jnp.dot(h1, wc, preferred_element_type=f32) + cb
    be = jax.nn.softplus(z)
    gh = jnp.maximum(sg[:, :H] + dg[:, :H], 0.0)
    y = jnp.dot(gh, dgW2[...], preferred_element_type=f32) + dgb2[...]
    ge = jax.nn.sigmoid(y)
    msg = (be * ge) * sg[:, 2 * H:2 * H + MSG]
    for q in range(NPASS):
        outs[q][...] = msg[:, 8 * q:8 * q + 8]
    npair = dstev.shape[0]
    de = dstev[...]
    do = dstod[...]
    hi = lax.broadcasted_iota(jnp.int32, (npair, 16), 1) >= 8
    col = lax.broadcasted_iota(jnp.int32, (npair, 16), 1) - 8 * hi.astype(
        jnp.int32)
    tgt_out[...] = jnp.where(hi, do * 8, de * 8) + col
    cm_out[...] = (hi & (de == do)).astype(jnp.int32)


def _edge_math(sg, dg, ea, dstev, dstod,
               esW1, esb1, esW2, esb2, bwW, bwb, dgW2, dgb2):
    be_blk = 1600
    npair = be_blk // 2
    grid = (E // be_blk,)
    row_spec = lambda w: pl.BlockSpec((be_blk, w), lambda i: (i, 0))
    full = lambda arr: pl.BlockSpec(arr.shape, lambda i: (0,) * arr.ndim)
    pair_spec = pl.BlockSpec((npair, 1), lambda i: (i, 0))
    pout_spec = pl.BlockSpec((npair, 16), lambda i: (i, 0))
    return pl.pallas_call(
        _edge_math_body,
        grid=grid,
        in_specs=[row_spec(SRCW), row_spec(DSTW), row_spec(DE),
                  pair_spec, pair_spec,
                  full(esW1), full(esb1), full(esW2), full(esb2),
                  full(bwW), full(bwb), full(dgW2), full(dgb2)],
        out_specs=[pout_spec, pout_spec] + [row_spec(8)] * NPASS,
        out_shape=[jax.ShapeDtypeStruct((E // 2, 16), jnp.int32),
                   jax.ShapeDtypeStruct((E // 2, 16), jnp.int32)]
        + [jax.ShapeDtypeStruct((E, 8), jnp.float32)] * NPASS,
        compiler_params=pltpu.CompilerParams(
            dimension_semantics=("arbitrary",)),
    )(sg, dg, ea, dstev, dstod,
      esW1, esb1, esW2, esb2, bwW, bwb, dgW2, dgb2)


CH2 = 1000
NCH2 = EPW // CH2
NPASS = MSG // 8


@functools.partial(
    pl.kernel,
    out_type=jax.ShapeDtypeStruct((NW, NPASS, NPAD * 8), jnp.float32),
    mesh=_mesh,
    scratch_types=[pltpu.VMEM((CH2 * 8,), jnp.int32),
                   pltpu.VMEM((CH2 * 8,), jnp.int32),
                   pltpu.VMEM((CH2 * 8,), jnp.float32),
                   pltpu.VMEM((NPAD * 8,), jnp.float32)],
    compiler_params=pltpu.CompilerParams(needs_layout_passes=False),
)
def _scatter_kernel(m0, m1, m2, m3, m4, m5, m6, m7, tgt_hbm, cm_hbm,
                    zeros_hbm, out_hbm, tgt_v, cm_v, msg_v, acc):
    cid = lax.axis_index("c")
    sid = lax.axis_index("s")
    wid = cid * NS + sid
    base0 = wid * EPW
    msgs = (m0, m1, m2, m3, m4, m5, m6, m7)
    npair = CH2 // 2

    for p in range(NPASS):
        pltpu.sync_copy(zeros_hbm, acc)

        def chunk_body(c, carry, _mp=msgs[p]):
            base = base0 + c * CH2
            pltpu.sync_copy(tgt_hbm.at[pl.ds(base * 8, CH2 * 8)], tgt_v)
            pltpu.sync_copy(cm_hbm.at[pl.ds(base * 8, CH2 * 8)], cm_v)
            pltpu.sync_copy(_mp.at[pl.ds(base * 8, CH2 * 8)], msg_v)

            @plsc.parallel_loop(0, npair, 1, unroll=8)
            def _pair_body(k):
                sl = pl.ds(16 * k, 16)
                tgt = tgt_v[sl]
                mask2 = cm_v[sl] > 0
                vals = msg_v[sl]
                plsc.addupdate_scatter(acc, [tgt], vals,
                                       mask=jnp.logical_not(mask2))
                plsc.addupdate_scatter(acc, [tgt], vals, mask=mask2)

            return carry

        lax.fori_loop(0, NCH2, chunk_body, 0)
        pltpu.sync_copy(acc, out_hbm.at[wid, p])


def _sum_body(p, out):
    out[...] = jnp.sum(p[...], axis=0)


def _sum_partials(partials):
    rows = NPASS * NPAD * 8 // 128
    p3 = partials.reshape(NW, rows, 128)
    blk = 512
    grid = (rows // blk,)
    red = pl.pallas_call(
        _sum_body,
        grid=grid,
        in_specs=[pl.BlockSpec((NW, blk, 128), lambda i: (0, i, 0))],
        out_specs=pl.BlockSpec((blk, 128), lambda i: (i, 0)),
        out_shape=jax.ShapeDtypeStruct((rows, 128), jnp.float32),
        compiler_params=pltpu.CompilerParams(
            dimension_semantics=("arbitrary",)),
    )(p3)
    r = red.reshape(NPASS, NPAD, 8)
    return jnp.transpose(r, (1, 0, 2)).reshape(NPAD, MSG)[:N]


def kernel(h_src, h_dst, edge_attr_static, x_static_src, x_static_dst,
           edge_index,
           es_W1, es_b1, es_W2, es_b2,
           bw_W, bw_b,
           dg_W1, dg_b1, dg_W2, dg_b2,
           pl_W1, pl_b1, pl_W2, pl_b2):
    f32 = jnp.float32
    src = edge_index[0].astype(jnp.int32)
    dst = edge_index[1].astype(jnp.int32)
    dgb1 = dg_b1.reshape(1, HID).astype(f32)
    esb1 = es_b1.reshape(1, HID).astype(f32)
    esb2 = es_b2.reshape(1, HID).astype(f32)
    plb1 = pl_b1.reshape(1, HID).astype(f32)
    plb2 = pl_b2.reshape(1, MSG).astype(f32)
    bwb = bw_b.reshape(1, 1).astype(f32)
    dgb2 = dg_b2.reshape(1, 1).astype(f32)

    srctab, dsttab = _node_tables(h_src, h_dst, x_static_src, x_static_dst,
                                  dg_W1, dgb1, es_W1, pl_W1, plb1, pl_W2, plb2)
    sg, dg = _gather_kernel(srctab, dsttab, src, dst)
    dstev = dst[0::2].reshape(E // 2, 1)
    dstod = dst[1::2].reshape(E // 2, 1)
    tgt16, cm16, *msgs = _edge_math(sg, dg, edge_attr_static, dstev, dstod,
                                    es_W1, esb1, es_W2, esb2, bw_W, bwb,
                                    dg_W2, dgb2)
    zeros = jnp.zeros((NPAD * 8,), f32)
    partials = _scatter_kernel(*[m.reshape(E * 8) for m in msgs],
                               tgt16.reshape(E * 8), cm16.reshape(E * 8),
                               zeros)
    return _sum_partials(partials.reshape(NW, NPASS * NPAD * 8))
